# baseline (speedup 1.0000x reference)
import functools

import jax
import jax.numpy as jnp
from jax import lax
from jax.experimental import pallas as pl
from jax.experimental.pallas import tpu as pltpu
from jax.experimental.pallas import tpu_sc as plsc

B = 16384
D = 64
W = 128
NC = 2
NS = 16
L = 16
NW = NC * NS
B_W = B // NW
NCHUNK = 4
CB = B_W // NCHUNK


def _kge_body(s_hbm, p_hbm, o_hbm, ent_hbm, rel_hbm, out_hbm,
              s_idx, p_idx, o_idx, s_gi, p_gi, o_gi,
              s_buf, p_buf, o_buf, out_v, tmat, sems):
    wid = lax.axis_index("s") * NC + lax.axis_index("c")
    base = wid * NCHUNK

    pltpu.sync_copy(s_hbm.at[pl.ds(base, NCHUNK)], s_idx)
    pltpu.sync_copy(p_hbm.at[pl.ds(base, NCHUNK)], p_idx)
    pltpu.sync_copy(o_hbm.at[pl.ds(base, NCHUNK)], o_idx)

    for c in range(NCHUNK):
        for k in range(CB // L):
            sl = pl.ds(k * L, L)
            s_gi[c, sl] = jax.lax.shift_right_logical(s_idx[c, sl], 1)
            p_gi[c, sl] = jax.lax.shift_right_logical(p_idx[c, sl], 1)
            o_gi[c, sl] = jax.lax.shift_right_logical(o_idx[c, sl], 1)

    def fire(c):
        slot = c % 2
        return (
            pltpu.async_copy(ent_hbm.at[s_gi.at[c]], s_buf.at[slot], sems.at[slot]),
            pltpu.async_copy(rel_hbm.at[p_gi.at[c]], p_buf.at[slot], sems.at[slot]),
            pltpu.async_copy(ent_hbm.at[o_gi.at[c]], o_buf.at[slot], sems.at[slot]),
        )

    lane17 = lax.iota(jnp.int32, L) * 17

    def compute(c):
        slot = c % 2

        def group_body(g, _):
            gsl = pl.ds(g * L, L)
            bs_v = (s_idx[c, gsl] & 1) * D
            bp_v = (p_idx[c, gsl] & 1) * D
            bo_v = (o_idx[c, gsl] & 1) * D
            for l in range(L):
                j = g * L + l
                bs = bs_v[l]
                bp = bp_v[l]
                bo = bo_v[l]
                acc = (s_buf[slot, j, pl.ds(bs, L)]
                       * p_buf[slot, j, pl.ds(bp, L)]
                       * o_buf[slot, j, pl.ds(bo, L)])
                for k in range(1, D // L):
                    acc = acc + (s_buf[slot, j, pl.ds(bs + k * L, L)]
                                 * p_buf[slot, j, pl.ds(bp + k * L, L)]
                                 * o_buf[slot, j, pl.ds(bo + k * L, L)])
                plsc.store_scatter(tmat, [lane17 + l], acc)
            res = tmat[pl.ds(0, L)]
            for d in range(1, L):
                res = res + tmat[pl.ds(d * 17, L)]
            out_v[pl.ds(c * CB + g * L, L)] = res
            return ()

        lax.fori_loop(0, CB // L, group_body, ())

    inflight = fire(0)
    for c in range(NCHUNK):
        nxt = fire(c + 1) if c + 1 < NCHUNK else None
        for cp in inflight:
            cp.wait()
        compute(c)
        inflight = nxt

    pltpu.sync_copy(out_v, out_hbm.at[wid])


def kernel(s, p, o, entity_emb, relation_emb):
    s2 = s.reshape(NW * NCHUNK, CB)
    p2 = p.reshape(NW * NCHUNK, CB)
    o2 = o.reshape(NW * NCHUNK, CB)
    ent2 = entity_emb.reshape(-1, W)
    rel2 = relation_emb.reshape(-1, W)
    mesh = plsc.VectorSubcoreMesh(core_axis_name="c", subcore_axis_name="s")
    run = pl.kernel(
        _kge_body,
        mesh=mesh,
        out_type=jax.ShapeDtypeStruct((NW, B_W), jnp.float32),
        compiler_params=pltpu.CompilerParams(needs_layout_passes=False),
        scratch_types=[
            pltpu.VMEM((NCHUNK, CB), jnp.int32),
            pltpu.VMEM((NCHUNK, CB), jnp.int32),
            pltpu.VMEM((NCHUNK, CB), jnp.int32),
            pltpu.VMEM((NCHUNK, CB), jnp.int32),
            pltpu.VMEM((NCHUNK, CB), jnp.int32),
            pltpu.VMEM((NCHUNK, CB), jnp.int32),
            pltpu.VMEM((2, CB, W), jnp.float32),
            pltpu.VMEM((2, CB, W), jnp.float32),
            pltpu.VMEM((2, CB, W), jnp.float32),
            pltpu.VMEM((B_W,), jnp.float32),
            pltpu.VMEM((L * 17,), jnp.float32),
            pltpu.SemaphoreType.DMA((2,)),
        ],
    )
    out = run(s2, p2, o2, ent2, rel2)
    return out.reshape(B, 1)


# speedup vs baseline: 1.1140x; 1.1140x over previous
import functools

import jax
import jax.numpy as jnp
from jax import lax
from jax.experimental import pallas as pl
from jax.experimental.pallas import tpu as pltpu
from jax.experimental.pallas import tpu_sc as plsc

B = 16384
D = 64
W = 128
NC = 2
NS = 16
L = 16
NW = NC * NS
B_W = B // NW
NCHUNK = 4
CB = B_W // NCHUNK


def _kge_body(s_hbm, p_hbm, o_hbm, ent_hbm, rel_hbm, out_hbm,
              s_idx, p_idx, o_idx,
              s_buf, p_buf, o_buf, out_v, tmat, sems):
    wid = lax.axis_index("s") * NC + lax.axis_index("c")
    base = wid * NCHUNK

    pltpu.sync_copy(s_hbm.at[pl.ds(base, NCHUNK)], s_idx)
    pltpu.sync_copy(p_hbm.at[pl.ds(base, NCHUNK)], p_idx)
    pltpu.sync_copy(o_hbm.at[pl.ds(base, NCHUNK)], o_idx)

    def fire(c):
        slot = c % 2
        return (
            pltpu.async_copy(ent_hbm.at[s_idx.at[c]], s_buf.at[slot], sems.at[slot]),
            pltpu.async_copy(rel_hbm.at[p_idx.at[c]], p_buf.at[slot], sems.at[slot]),
            pltpu.async_copy(ent_hbm.at[o_idx.at[c]], o_buf.at[slot], sems.at[slot]),
        )

    lane17 = lax.iota(jnp.int32, L) * 17

    def compute(c):
        slot = c % 2

        def group_body(g, _):
            for l in range(L):
                j = g * L + l
                acc = (s_buf[slot, j, pl.ds(0, L)]
                       * p_buf[slot, j, pl.ds(0, L)]
                       * o_buf[slot, j, pl.ds(0, L)])
                for k in range(1, D // L):
                    sl = pl.ds(k * L, L)
                    acc = acc + (s_buf[slot, j, sl]
                                 * p_buf[slot, j, sl]
                                 * o_buf[slot, j, sl])
                plsc.store_scatter(tmat, [lane17 + l], acc)
            res = tmat[pl.ds(0, L)]
            for d in range(1, L):
                res = res + tmat[pl.ds(d * 17, L)]
            out_v[pl.ds(c * CB + g * L, L)] = res
            return ()

        lax.fori_loop(0, CB // L, group_body, ())

    inflight = fire(0)
    for c in range(NCHUNK):
        nxt = fire(c + 1) if c + 1 < NCHUNK else None
        for cp in inflight:
            cp.wait()
        compute(c)
        inflight = nxt

    pltpu.sync_copy(out_v, out_hbm.at[wid])


def kernel(s, p, o, entity_emb, relation_emb):
    s2 = s.reshape(NW * NCHUNK, CB)
    p2 = p.reshape(NW * NCHUNK, CB)
    o2 = o.reshape(NW * NCHUNK, CB)
    ent2 = jnp.pad(entity_emb, ((0, 0), (0, W - D)))
    rel2 = jnp.pad(relation_emb, ((0, 0), (0, W - D)))
    mesh = plsc.VectorSubcoreMesh(core_axis_name="c", subcore_axis_name="s")
    run = pl.kernel(
        _kge_body,
        mesh=mesh,
        out_type=jax.ShapeDtypeStruct((NW, B_W), jnp.float32),
        compiler_params=pltpu.CompilerParams(needs_layout_passes=False),
        scratch_types=[
            pltpu.VMEM((NCHUNK, CB), jnp.int32),
            pltpu.VMEM((NCHUNK, CB), jnp.int32),
            pltpu.VMEM((NCHUNK, CB), jnp.int32),
            pltpu.VMEM((2, CB, W), jnp.float32),
            pltpu.VMEM((2, CB, W), jnp.float32),
            pltpu.VMEM((2, CB, W), jnp.float32),
            pltpu.VMEM((B_W,), jnp.float32),
            pltpu.VMEM((L * 17,), jnp.float32),
            pltpu.SemaphoreType.DMA((2,)),
        ],
    )
    out = run(s2, p2, o2, ent2, rel2)
    return out.reshape(B, 1)


# speedup vs baseline: 1.4845x; 1.3326x over previous
import functools

import jax
import jax.numpy as jnp
from jax import lax
from jax.experimental import pallas as pl
from jax.experimental.pallas import tpu as pltpu
from jax.experimental.pallas import tpu_sc as plsc

B = 16384
D = 64
W = 128
G = 8
NC = 2
NS = 16
L = 16
NW = NC * NS
B_W = B // NW
NCHUNK = 32
CB = B_W // NCHUNK


def _kge_body(s_hbm, p_hbm, o_hbm, ent_hbm, rel_hbm, out_hbm,
              s_idx, p_idx, o_idx,
              s_buf, p_buf, o_buf, out_v, tmat, sems):
    wid = lax.axis_index("s") * NC + lax.axis_index("c")
    base = wid * NCHUNK

    pltpu.sync_copy(s_hbm.at[pl.ds(base, NCHUNK)], s_idx)
    pltpu.sync_copy(p_hbm.at[pl.ds(base, NCHUNK)], p_idx)
    pltpu.sync_copy(o_hbm.at[pl.ds(base, NCHUNK)], o_idx)

    def fire(c):
        slot = c % 2
        pltpu.async_copy(rel_hbm.at[p_idx.at[c]], p_buf.at[slot], sems.at[slot])
        gsl = pl.ds(0, L)
        sg_v = jax.lax.shift_right_logical(s_idx[c, gsl], 3) * G
        og_v = jax.lax.shift_right_logical(o_idx[c, gsl], 3) * G
        for l in range(L):
            sg = pl.multiple_of(sg_v[l], G)
            og = pl.multiple_of(og_v[l], G)
            pltpu.async_copy(ent_hbm.at[pl.ds(sg, G)],
                             s_buf.at[slot, pl.ds(l * G, G)],
                             sems.at[slot])
            pltpu.async_copy(ent_hbm.at[pl.ds(og, G)],
                             o_buf.at[slot, pl.ds(l * G, G)],
                             sems.at[slot])

    def drain(c):
        slot = c % 2
        pltpu.make_async_copy(rel_hbm.at[pl.ds(0, CB)], p_buf.at[slot],
                              sems.at[slot]).wait()
        pltpu.make_async_copy(ent_hbm.at[pl.ds(0, CB * G)], s_buf.at[slot],
                              sems.at[slot]).wait()
        pltpu.make_async_copy(ent_hbm.at[pl.ds(0, CB * G)], o_buf.at[slot],
                              sems.at[slot]).wait()

    lane17 = lax.iota(jnp.int32, L) * 17

    def compute(c):
        slot = c % 2
        gsl = pl.ds(0, L)
        ss_v = s_idx[c, gsl] & (G - 1)
        so_v = o_idx[c, gsl] & (G - 1)
        for l in range(L):
            ss = l * G + ss_v[l]
            so = l * G + so_v[l]
            acc = (s_buf[slot, ss, pl.ds(0, L)]
                   * p_buf[slot, l, pl.ds(0, L)]
                   * o_buf[slot, so, pl.ds(0, L)])
            for k in range(1, D // L):
                sl = pl.ds(k * L, L)
                acc = acc + (s_buf[slot, ss, sl]
                             * p_buf[slot, l, sl]
                             * o_buf[slot, so, sl])
            plsc.store_scatter(tmat, [lane17 + l], acc)
        res = tmat[pl.ds(0, L)]
        for d in range(1, L):
            res = res + tmat[pl.ds(d * 17, L)]
        out_v[pl.ds(c * CB, L)] = res

    fire(0)

    def step(c, _):
        @pl.when(c + 1 < NCHUNK)
        def _():
            fire(c + 1)

        drain(c)
        compute(c)
        return ()

    lax.fori_loop(0, NCHUNK, step, ())

    pltpu.sync_copy(out_v, out_hbm.at[wid])


def kernel(s, p, o, entity_emb, relation_emb):
    s2 = s.reshape(NW * NCHUNK, CB)
    p2 = p.reshape(NW * NCHUNK, CB)
    o2 = o.reshape(NW * NCHUNK, CB)
    rel2 = jnp.pad(relation_emb, ((0, 0), (0, W - D)))
    mesh = plsc.VectorSubcoreMesh(core_axis_name="c", subcore_axis_name="s")
    run = pl.kernel(
        _kge_body,
        mesh=mesh,
        out_type=jax.ShapeDtypeStruct((NW, B_W), jnp.float32),
        compiler_params=pltpu.CompilerParams(needs_layout_passes=False),
        scratch_types=[
            pltpu.VMEM((NCHUNK, CB), jnp.int32),
            pltpu.VMEM((NCHUNK, CB), jnp.int32),
            pltpu.VMEM((NCHUNK, CB), jnp.int32),
            pltpu.VMEM((2, CB * G, D), jnp.float32),
            pltpu.VMEM((2, CB, W), jnp.float32),
            pltpu.VMEM((2, CB * G, D), jnp.float32),
            pltpu.VMEM((B_W,), jnp.float32),
            pltpu.VMEM((L * 17,), jnp.float32),
            pltpu.SemaphoreType.DMA((2,)),
        ],
    )
    out = run(s2, p2, o2, entity_emb, rel2)
    return out.reshape(B, 1)


# speedup vs baseline: 2.1473x; 1.4465x over previous
import functools

import jax
import jax.numpy as jnp
from jax import lax
from jax.experimental import pallas as pl
from jax.experimental.pallas import tpu as pltpu
from jax.experimental.pallas import tpu_sc as plsc

B = 16384
D = 64
W = 128
G = 8
NC = 2
NS = 16
L = 16
NW = NC * NS
B_W = B // NW
NCHUNK = 32
CB = B_W // NCHUNK


def _kge_body(s_hbm, p_hbm, o_hbm, ent_hbm, rel_hbm, out_hbm,
              s_idx, p_idx, o_idx,
              s_buf, p_buf, o_buf, out_v, tmat, sems):
    wid = lax.axis_index("s") * NC + lax.axis_index("c")
    base = wid * NCHUNK

    pltpu.sync_copy(s_hbm.at[pl.ds(base, NCHUNK)], s_idx)
    pltpu.sync_copy(p_hbm.at[pl.ds(base, NCHUNK)], p_idx)
    pltpu.sync_copy(o_hbm.at[pl.ds(base, NCHUNK)], o_idx)

    def fire(c):
        slot = c % 2
        pltpu.async_copy(rel_hbm.at[p_idx.at[c]], p_buf.at[slot], sems.at[slot])
        gsl = pl.ds(0, L)
        sg_v = jax.lax.shift_right_logical(s_idx[c, gsl], 3)
        og_v = jax.lax.shift_right_logical(o_idx[c, gsl], 3)
        for l in range(L):
            pltpu.async_copy(ent_hbm.at[sg_v[l]], s_buf.at[slot, l],
                             sems.at[slot])
            pltpu.async_copy(ent_hbm.at[og_v[l]], o_buf.at[slot, l],
                             sems.at[slot])

    def drain(c):
        slot = c % 2
        pltpu.make_async_copy(rel_hbm.at[pl.ds(0, CB)], p_buf.at[slot],
                              sems.at[slot]).wait()
        pltpu.make_async_copy(ent_hbm.at[pl.ds(0, CB)], s_buf.at[slot],
                              sems.at[slot]).wait()
        pltpu.make_async_copy(ent_hbm.at[pl.ds(0, CB)], o_buf.at[slot],
                              sems.at[slot]).wait()

    lane17 = lax.iota(jnp.int32, L) * 17

    def compute(c):
        slot = c % 2
        gsl = pl.ds(0, L)
        ss_v = s_idx[c, gsl] & (G - 1)
        so_v = o_idx[c, gsl] & (G - 1)
        for l in range(L):
            ss = ss_v[l]
            so = so_v[l]
            acc = (s_buf[slot, l, ss, pl.ds(0, L)]
                   * p_buf[slot, l, pl.ds(0, L)]
                   * o_buf[slot, l, so, pl.ds(0, L)])
            for k in range(1, D // L):
                sl = pl.ds(k * L, L)
                acc = acc + (s_buf[slot, l, ss, sl]
                             * p_buf[slot, l, sl]
                             * o_buf[slot, l, so, sl])
            plsc.store_scatter(tmat, [lane17 + l], acc)
        res = tmat[pl.ds(0, L)]
        for d in range(1, L):
            res = res + tmat[pl.ds(d * 17, L)]
        out_v[pl.ds(c * CB, L)] = res

    fire(0)

    def step(c, _):
        @pl.when(c + 1 < NCHUNK)
        def _():
            fire(c + 1)

        drain(c)
        compute(c)
        return ()

    lax.fori_loop(0, NCHUNK, step, ())

    pltpu.sync_copy(out_v, out_hbm.at[wid])


def kernel(s, p, o, entity_emb, relation_emb):
    s2 = s.reshape(NW * NCHUNK, CB)
    p2 = p.reshape(NW * NCHUNK, CB)
    o2 = o.reshape(NW * NCHUNK, CB)
    rel2 = jnp.pad(relation_emb, ((0, 0), (0, W - D)))
    ent4 = entity_emb.reshape(-1, G, D)
    mesh = plsc.VectorSubcoreMesh(core_axis_name="c", subcore_axis_name="s")
    run = pl.kernel(
        _kge_body,
        mesh=mesh,
        out_type=jax.ShapeDtypeStruct((NW, B_W), jnp.float32),
        compiler_params=pltpu.CompilerParams(needs_layout_passes=False),
        scratch_types=[
            pltpu.VMEM((NCHUNK, CB), jnp.int32),
            pltpu.VMEM((NCHUNK, CB), jnp.int32),
            pltpu.VMEM((NCHUNK, CB), jnp.int32),
            pltpu.VMEM((2, CB, G, D), jnp.float32),
            pltpu.VMEM((2, CB, W), jnp.float32),
            pltpu.VMEM((2, CB, G, D), jnp.float32),
            pltpu.VMEM((B_W,), jnp.float32),
            pltpu.VMEM((L * 17,), jnp.float32),
            pltpu.SemaphoreType.DMA((2,)),
        ],
    )
    out = run(s2, p2, o2, ent4, rel2)
    return out.reshape(B, 1)


# speedup vs baseline: 2.1921x; 1.0209x over previous
import functools

import jax
import jax.numpy as jnp
from jax import lax
from jax.experimental import pallas as pl
from jax.experimental.pallas import tpu as pltpu
from jax.experimental.pallas import tpu_sc as plsc

B = 16384
D = 64
W = 128
G = 8
NC = 2
NS = 16
L = 16
NW = NC * NS
B_W = B // NW
NCHUNK = 32
CB = B_W // NCHUNK
NBUF = 3


def _kge_body(s_hbm, p_hbm, o_hbm, ent_hbm, rel_hbm, out_hbm,
              s_idx, p_idx, o_idx,
              s_buf, p_buf, o_buf, out_v, tmat, sems):
    wid = lax.axis_index("s") * NC + lax.axis_index("c")
    base = wid * NCHUNK

    pltpu.sync_copy(s_hbm.at[pl.ds(base, NCHUNK)], s_idx)
    pltpu.sync_copy(p_hbm.at[pl.ds(base, NCHUNK)], p_idx)
    pltpu.sync_copy(o_hbm.at[pl.ds(base, NCHUNK)], o_idx)

    def fire(c):
        slot = c % NBUF
        pltpu.async_copy(rel_hbm.at[p_idx.at[c]], p_buf.at[slot], sems.at[slot])
        gsl = pl.ds(0, L)
        sg_v = jax.lax.shift_right_logical(s_idx[c, gsl], 3)
        og_v = jax.lax.shift_right_logical(o_idx[c, gsl], 3)
        for l in range(L):
            pltpu.async_copy(ent_hbm.at[sg_v[l]], s_buf.at[slot, l],
                             sems.at[slot])
            pltpu.async_copy(ent_hbm.at[og_v[l]], o_buf.at[slot, l],
                             sems.at[slot])

    def drain(c):
        slot = c % NBUF
        pltpu.make_async_copy(rel_hbm.at[pl.ds(0, CB)], p_buf.at[slot],
                              sems.at[slot]).wait()
        pltpu.make_async_copy(ent_hbm.at[pl.ds(0, CB)], s_buf.at[slot],
                              sems.at[slot]).wait()
        pltpu.make_async_copy(ent_hbm.at[pl.ds(0, CB)], o_buf.at[slot],
                              sems.at[slot]).wait()

    lane17 = lax.iota(jnp.int32, L) * 17

    def compute(c):
        slot = c % NBUF
        gsl = pl.ds(0, L)
        ss_v = s_idx[c, gsl] & (G - 1)
        so_v = o_idx[c, gsl] & (G - 1)
        for l in range(L):
            ss = ss_v[l]
            so = so_v[l]
            acc = (s_buf[slot, l, ss, pl.ds(0, L)]
                   * p_buf[slot, l, pl.ds(0, L)]
                   * o_buf[slot, l, so, pl.ds(0, L)])
            for k in range(1, D // L):
                sl = pl.ds(k * L, L)
                acc = acc + (s_buf[slot, l, ss, sl]
                             * p_buf[slot, l, sl]
                             * o_buf[slot, l, so, sl])
            plsc.store_scatter(tmat, [lane17 + l], acc)
        res = tmat[pl.ds(0, L)]
        for d in range(1, L):
            res = res + tmat[pl.ds(d * 17, L)]
        out_v[pl.ds(c * CB, L)] = res

    for k in range(NBUF - 1):
        fire(k)

    def step(c, _):
        @pl.when(c + NBUF - 1 < NCHUNK)
        def _():
            fire(c + NBUF - 1)

        drain(c)
        compute(c)
        return ()

    lax.fori_loop(0, NCHUNK, step, ())

    pltpu.sync_copy(out_v, out_hbm.at[wid])


def kernel(s, p, o, entity_emb, relation_emb):
    s2 = s.reshape(NW * NCHUNK, CB)
    p2 = p.reshape(NW * NCHUNK, CB)
    o2 = o.reshape(NW * NCHUNK, CB)
    rel2 = jnp.pad(relation_emb, ((0, 0), (0, W - D)))
    ent4 = entity_emb.reshape(-1, G, D)
    mesh = plsc.VectorSubcoreMesh(core_axis_name="c", subcore_axis_name="s")
    run = pl.kernel(
        _kge_body,
        mesh=mesh,
        out_type=jax.ShapeDtypeStruct((NW, B_W), jnp.float32),
        compiler_params=pltpu.CompilerParams(needs_layout_passes=False),
        scratch_types=[
            pltpu.VMEM((NCHUNK, CB), jnp.int32),
            pltpu.VMEM((NCHUNK, CB), jnp.int32),
            pltpu.VMEM((NCHUNK, CB), jnp.int32),
            pltpu.VMEM((NBUF, CB, G, D), jnp.float32),
            pltpu.VMEM((NBUF, CB, W), jnp.float32),
            pltpu.VMEM((NBUF, CB, G, D), jnp.float32),
            pltpu.VMEM((B_W,), jnp.float32),
            pltpu.VMEM((L * 17,), jnp.float32),
            pltpu.SemaphoreType.DMA((NBUF,)),
        ],
    )
    out = run(s2, p2, o2, ent4, rel2)
    return out.reshape(B, 1)


# speedup vs baseline: 2.1932x; 1.0005x over previous
import jax
import jax.numpy as jnp
from jax import lax
from jax.experimental import pallas as pl
from jax.experimental.pallas import tpu as pltpu
from jax.experimental.pallas import tpu_sc as plsc

B = 16384
D = 64
W = 128
G = 8
NC = 2
NS = 16
L = 16
NW = NC * NS
B_W = B // NW
NCHUNK = 32
CB = B_W // NCHUNK
NBUF = 3


def _kge_body(s_hbm, p_hbm, o_hbm, ent_hbm, rel_hbm, out_hbm,
              s_idx, p_idx, o_idx,
              s_buf, p_buf, o_buf, out_v, tmat, sems):
    wid = lax.axis_index("s") * NC + lax.axis_index("c")
    base = wid * NCHUNK

    pltpu.sync_copy(s_hbm.at[pl.ds(base, NCHUNK)], s_idx)
    pltpu.sync_copy(p_hbm.at[pl.ds(base, NCHUNK)], p_idx)
    pltpu.sync_copy(o_hbm.at[pl.ds(base, NCHUNK)], o_idx)

    def fire(c):
        slot = c % NBUF
        pltpu.async_copy(rel_hbm.at[p_idx.at[c]], p_buf.at[slot], sems.at[slot])
        gsl = pl.ds(0, L)
        sg_v = jax.lax.shift_right_logical(s_idx[c, gsl], 3)
        og_v = jax.lax.shift_right_logical(o_idx[c, gsl], 3)
        for l in range(L):
            pltpu.async_copy(ent_hbm.at[sg_v[l]], s_buf.at[slot, l],
                             sems.at[slot])
            pltpu.async_copy(ent_hbm.at[og_v[l]], o_buf.at[slot, l],
                             sems.at[slot])

    def drain(c):
        slot = c % NBUF
        pltpu.make_async_copy(rel_hbm.at[pl.ds(0, CB)], p_buf.at[slot],
                              sems.at[slot]).wait()
        pltpu.make_async_copy(ent_hbm.at[pl.ds(0, CB)], s_buf.at[slot],
                              sems.at[slot]).wait()
        pltpu.make_async_copy(ent_hbm.at[pl.ds(0, CB)], o_buf.at[slot],
                              sems.at[slot]).wait()

    lane17 = lax.iota(jnp.int32, L) * 17

    def compute(c):
        slot = c % NBUF
        gsl = pl.ds(0, L)
        ss_v = s_idx[c, gsl] & (G - 1)
        so_v = o_idx[c, gsl] & (G - 1)
        for l in range(L):
            ss = ss_v[l]
            so = so_v[l]
            acc = (s_buf[slot, l, ss, pl.ds(0, L)]
                   * p_buf[slot, l, pl.ds(0, L)]
                   * o_buf[slot, l, so, pl.ds(0, L)])
            for k in range(1, D // L):
                sl = pl.ds(k * L, L)
                acc = acc + (s_buf[slot, l, ss, sl]
                             * p_buf[slot, l, sl]
                             * o_buf[slot, l, so, sl])
            plsc.store_scatter(tmat, [lane17 + l], acc)
        res = tmat[pl.ds(0, L)]
        for d in range(1, L):
            res = res + tmat[pl.ds(d * 17, L)]
        out_v[pl.ds(c * CB, L)] = res

    for k in range(NBUF - 1):
        fire(k)

    def step(c, _):
        @pl.when(c + NBUF - 1 < NCHUNK)
        def _():
            fire(c + NBUF - 1)

        drain(c)
        compute(c)
        return ()

    lax.fori_loop(0, NCHUNK, step, ())

    pltpu.sync_copy(out_v, out_hbm.at[wid])


def kernel(s, p, o, entity_emb, relation_emb):
    s2 = s.reshape(NW * NCHUNK, CB)
    p2 = p.reshape(NW * NCHUNK, CB)
    o2 = o.reshape(NW * NCHUNK, CB)
    rel2 = jnp.pad(relation_emb, ((0, 0), (0, W - D)))
    ent4 = entity_emb.reshape(-1, G, D)
    mesh = plsc.VectorSubcoreMesh(core_axis_name="c", subcore_axis_name="s")
    run = pl.kernel(
        _kge_body,
        mesh=mesh,
        out_type=jax.ShapeDtypeStruct((NW, B_W), jnp.float32),
        compiler_params=pltpu.CompilerParams(needs_layout_passes=False),
        scratch_types=[
            pltpu.VMEM((NCHUNK, CB), jnp.int32),
            pltpu.VMEM((NCHUNK, CB), jnp.int32),
            pltpu.VMEM((NCHUNK, CB), jnp.int32),
            pltpu.VMEM((NBUF, CB, G, D), jnp.float32),
            pltpu.VMEM((NBUF, CB, W), jnp.float32),
            pltpu.VMEM((NBUF, CB, G, D), jnp.float32),
            pltpu.VMEM((B_W,), jnp.float32),
            pltpu.VMEM((L * 17,), jnp.float32),
            pltpu.SemaphoreType.DMA((NBUF,)),
        ],
    )
    out = run(s2, p2, o2, ent4, rel2)
    return out.reshape(B, 1)
